# Initial kernel scaffold; baseline (speedup 1.0000x reference)
import jax
import jax.numpy as jnp
from jax.experimental import pallas as pl


def kernel(x_customer, x_product, edge_index, lin_c_w, lin_c_b, lin_p_w, lin_p_b, w_self, w_neigh, b_graph):
    raise NotImplementedError("write your pallas kernel here")


# speedup vs baseline: 5.9829x; 5.9829x over previous
import jax
import jax.numpy as jnp
from jax import lax
from jax.experimental import pallas as pl
from jax.experimental.pallas import tpu as pltpu
from jax.experimental.pallas import tpu_sc as plsc

N_NODES_K = 10000
N_OUT_K = 6000
GDIM = 128
E_K = 320000
NC_K, NS_K = 2, 16
NW_K = NC_K * NS_K
CHUNK_K = 80
EDGES_PER_TILE_K = E_K // NW_K
N_CHUNKS_K = EDGES_PER_TILE_K // CHUNK_K
SP_ROWS_K = 10240
STRIPE_K = SP_ROWS_K // NS_K
N_PAD_K = 6016
OUT_STRIPE_K = N_PAD_K // NS_K


def _proj_body(x_ref, w_ref, b_ref, o_ref):
    o_ref[...] = jnp.dot(x_ref[...], w_ref[...],
                         preferred_element_type=jnp.float32) + b_ref[...]


def _proj(x, w, b, block):
    m = x.shape[0]
    return pl.pallas_call(
        _proj_body,
        grid=(m // block,),
        in_specs=[
            pl.BlockSpec((block, GDIM), lambda i: (i, 0)),
            pl.BlockSpec((GDIM, GDIM), lambda i: (0, 0)),
            pl.BlockSpec((1, GDIM), lambda i: (0, 0)),
        ],
        out_specs=pl.BlockSpec((block, GDIM), lambda i: (i, 0)),
        out_shape=jax.ShapeDtypeStruct((m, GDIM), jnp.float32),
    )(x, w, b.reshape(1, GDIM))


def _sc_body(x_hbm, src_hbm, dst_hbm, agg_out, deg_out,
             src_v, dst_v, rows_v, deg_v, agg_sh, sem):
    c = lax.axis_index("c")
    s = lax.axis_index("s")
    wid = c * NS_K + s

    def zero_deg(i, carry):
        deg_v[pl.ds(i * 16, 16)] = jnp.zeros((16,), jnp.float32)
        return carry
    lax.fori_loop(0, N_NODES_K // 16, zero_deg, 0)

    def zero_rows(t, carry):
        rows_v[t // 8, pl.ds((t % 8) * 16, 16)] = jnp.zeros((16,), jnp.float32)
        return carry
    lax.fori_loop(0, CHUNK_K * 8, zero_rows, 0)
    base_row = s * STRIPE_K
    for j in range(STRIPE_K // CHUNK_K):
        pltpu.sync_copy(rows_v, agg_sh.at[pl.ds(base_row + j * CHUNK_K, CHUNK_K)])
    plsc.subcore_barrier()

    edge_base = wid * EDGES_PER_TILE_K
    ones16 = jnp.ones((16,), jnp.float32)

    def body(i, carry):
        b = edge_base + i * CHUNK_K
        pltpu.sync_copy(src_hbm.at[pl.ds(b, CHUNK_K)], src_v)
        pltpu.sync_copy(dst_hbm.at[pl.ds(b, CHUNK_K)], dst_v)
        pltpu.async_copy(x_hbm.at[src_v], rows_v, sem).wait()
        pltpu.sync_copy(rows_v, agg_sh.at[dst_v], add=True)
        for j in range(CHUNK_K // 16):
            d16 = dst_v[pl.ds(j * 16, 16)]
            plsc.addupdate_scatter(deg_v, [d16], ones16)
        return carry
    lax.fori_loop(0, N_CHUNKS_K, body, 0)
    plsc.subcore_barrier()

    out_base = s * OUT_STRIPE_K
    pltpu.sync_copy(agg_sh.at[pl.ds(out_base, OUT_STRIPE_K)],
                    agg_out.at[c, pl.ds(out_base, OUT_STRIPE_K)])
    pltpu.sync_copy(deg_v.at[pl.ds(0, N_PAD_K)], deg_out.at[wid])


def _sc_agg(x, edge_index):
    mesh = plsc.VectorSubcoreMesh(core_axis_name="c", subcore_axis_name="s")
    return pl.kernel(
        _sc_body,
        out_type=[
            jax.ShapeDtypeStruct((NC_K, N_PAD_K, GDIM), jnp.float32),
            jax.ShapeDtypeStruct((NW_K, N_PAD_K), jnp.float32),
        ],
        mesh=mesh,
        scratch_types=[
            pltpu.VMEM((CHUNK_K,), jnp.int32),
            pltpu.VMEM((CHUNK_K,), jnp.int32),
            pltpu.VMEM((CHUNK_K, GDIM), jnp.float32),
            pltpu.VMEM((N_NODES_K,), jnp.float32),
            pltpu.VMEM_SHARED((SP_ROWS_K, GDIM), jnp.float32),
            pltpu.SemaphoreType.DMA,
        ],
        compiler_params=pltpu.CompilerParams(needs_layout_passes=False),
    )(x, edge_index[0], edge_index[1])


def _final_body(x_ref, agg_ref, deg_ref, ws_ref, wn_ref, b_ref, o_ref):
    aggs = agg_ref[0] + agg_ref[1]
    deg = jnp.maximum(jnp.sum(deg_ref[...], axis=0), 1.0)
    agg = aggs / deg[:, None]
    res = jnp.maximum(
        jnp.dot(x_ref[...], ws_ref[...], preferred_element_type=jnp.float32)
        + jnp.dot(agg, wn_ref[...], preferred_element_type=jnp.float32)
        + b_ref[...], 0.0)
    o_ref[...] = res[:N_OUT_K]


def _final(x, agg2, deg32, w_self, w_neigh, b_graph):
    return pl.pallas_call(
        _final_body,
        out_shape=jax.ShapeDtypeStruct((N_OUT_K, GDIM), jnp.float32),
    )(x[:N_PAD_K], agg2, deg32, w_self, w_neigh, b_graph.reshape(1, GDIM))


def kernel(x_customer, x_product, edge_index, lin_c_w, lin_c_b,
           lin_p_w, lin_p_b, w_self, w_neigh, b_graph):
    h_c = _proj(x_customer, lin_c_w, lin_c_b, 1000)
    h_p = _proj(x_product, lin_p_w, lin_p_b, 1000)
    x = jnp.concatenate([h_c, h_p], axis=0)
    agg2, deg32 = _sc_agg(x, edge_index)
    return _final(x, agg2, deg32, w_self, w_neigh, b_graph)


# speedup vs baseline: 9.6937x; 1.6202x over previous
import jax
import jax.numpy as jnp
from jax import lax
from jax.experimental import pallas as pl
from jax.experimental.pallas import tpu as pltpu
from jax.experimental.pallas import tpu_sc as plsc

N_NODES_K = 10000
N_OUT_K = 6000
GDIM = 128
E_K = 320000
NC_K, NS_K = 2, 16
NW_K = NC_K * NS_K
CHUNK_K = 80
EDGES_PER_TILE_K = E_K // NW_K
N_CHUNKS_K = EDGES_PER_TILE_K // CHUNK_K
SP_ROWS_K = 6144
STRIPE_K = SP_ROWS_K // NS_K
TRASH_K = 6016
N_PAD_K = 6016
OUT_STRIPE_K = N_PAD_K // NS_K


def _proj_body(x_ref, w_ref, b_ref, o_ref):
    o_ref[...] = jnp.dot(x_ref[...], w_ref[...],
                         preferred_element_type=jnp.float32) + b_ref[...]


def _proj(x, w, b, block):
    m = x.shape[0]
    return pl.pallas_call(
        _proj_body,
        grid=(m // block,),
        in_specs=[
            pl.BlockSpec((block, GDIM), lambda i: (i, 0)),
            pl.BlockSpec((GDIM, GDIM), lambda i: (0, 0)),
            pl.BlockSpec((1, GDIM), lambda i: (0, 0)),
        ],
        out_specs=pl.BlockSpec((block, GDIM), lambda i: (i, 0)),
        out_shape=jax.ShapeDtypeStruct((m, GDIM), jnp.float32),
    )(x, w, b.reshape(1, GDIM))


def _sc_body(x_hbm, src_hbm, dst_hbm, agg_out, deg_out,
             src_b, dst_b, rows0, rows1, deg_v, agg_sh, g0, g1, s0, s1):
    c = lax.axis_index("c")
    s = lax.axis_index("s")
    wid = c * NS_K + s

    pltpu.sync_copy(src_hbm.at[wid], src_b)
    pltpu.sync_copy(dst_hbm.at[wid], dst_b)

    def zero_deg(i, carry):
        deg_v[pl.ds(i * 16, 16)] = jnp.zeros((16,), jnp.float32)
        return carry
    lax.fori_loop(0, SP_ROWS_K // 16, zero_deg, 0)

    def zero_rows(t, carry):
        rows0[t // 8, pl.ds((t % 8) * 16, 16)] = jnp.zeros((16,), jnp.float32)
        return carry
    lax.fori_loop(0, CHUNK_K * 8, zero_rows, 0)
    base_row = s * STRIPE_K
    for j in range(STRIPE_K // CHUNK_K):
        pltpu.sync_copy(rows0, agg_sh.at[pl.ds(base_row + j * CHUNK_K, CHUNK_K)])
    rem = STRIPE_K - (STRIPE_K // CHUNK_K) * CHUNK_K
    pltpu.sync_copy(rows0.at[pl.ds(0, rem)],
                    agg_sh.at[pl.ds(base_row + STRIPE_K - rem, rem)])
    plsc.subcore_barrier()

    ones16 = jnp.ones((16,), jnp.float32)

    def pre_body(t, carry):
        i = t // (CHUNK_K // 16)
        off = (t - i * (CHUNK_K // 16)) * 16
        d16 = dst_b[i, pl.ds(off, 16)]
        d16 = jnp.minimum(d16, TRASH_K)
        dst_b[i, pl.ds(off, 16)] = d16
        plsc.addupdate_scatter(deg_v, [d16], ones16)
        return carry
    lax.fori_loop(0, N_CHUNKS_K * (CHUNK_K // 16), pre_body, 0)

    def chunk_work(i, rows_p, g_p, s_p, rows_o, g_o, s_o):
        pltpu.make_async_copy(x_hbm.at[src_b.at[i]], rows_p, g_p).wait()
        pltpu.async_copy(rows_p, agg_sh.at[dst_b.at[i]], s_p, add=True)

        @pl.when(i >= 1)
        def _():
            pltpu.make_async_copy(rows_o, agg_sh.at[dst_b.at[i]], s_o).wait()

        @pl.when(i + 1 < N_CHUNKS_K)
        def _():
            pltpu.async_copy(x_hbm.at[src_b.at[i + 1]], rows_o, g_o)

    pltpu.async_copy(x_hbm.at[src_b.at[0]], rows0, g0)

    def body(i, carry):
        @pl.when(lax.rem(i, 2) == 0)
        def _():
            chunk_work(i, rows0, g0, s0, rows1, g1, s1)

        @pl.when(lax.rem(i, 2) == 1)
        def _():
            chunk_work(i, rows1, g1, s1, rows0, g0, s0)
        return carry
    lax.fori_loop(0, N_CHUNKS_K, body, 0)
    pltpu.make_async_copy(rows0, agg_sh.at[dst_b.at[0]], s0).wait()
    plsc.subcore_barrier()

    out_base = s * OUT_STRIPE_K
    pltpu.sync_copy(agg_sh.at[pl.ds(out_base, OUT_STRIPE_K)],
                    agg_out.at[c, pl.ds(out_base, OUT_STRIPE_K)])
    pltpu.sync_copy(deg_v.at[pl.ds(0, N_PAD_K)], deg_out.at[wid])


def _sc_agg(x, edge_index):
    mesh = plsc.VectorSubcoreMesh(core_axis_name="c", subcore_axis_name="s")
    idx3 = edge_index.reshape(2, NW_K, N_CHUNKS_K, CHUNK_K)
    return pl.kernel(
        _sc_body,
        out_type=[
            jax.ShapeDtypeStruct((NC_K, N_PAD_K, GDIM), jnp.float32),
            jax.ShapeDtypeStruct((NW_K, N_PAD_K), jnp.float32),
        ],
        mesh=mesh,
        scratch_types=[
            pltpu.VMEM((N_CHUNKS_K, CHUNK_K), jnp.int32),
            pltpu.VMEM((N_CHUNKS_K, CHUNK_K), jnp.int32),
            pltpu.VMEM((CHUNK_K, GDIM), jnp.float32),
            pltpu.VMEM((CHUNK_K, GDIM), jnp.float32),
            pltpu.VMEM((SP_ROWS_K,), jnp.float32),
            pltpu.VMEM_SHARED((SP_ROWS_K, GDIM), jnp.float32),
            pltpu.SemaphoreType.DMA,
            pltpu.SemaphoreType.DMA,
            pltpu.SemaphoreType.DMA,
            pltpu.SemaphoreType.DMA,
        ],
        compiler_params=pltpu.CompilerParams(needs_layout_passes=False),
    )(x, idx3[0], idx3[1])


def _final_body(x_ref, agg_ref, deg_ref, ws_ref, wn_ref, b_ref, o_ref):
    aggs = agg_ref[0] + agg_ref[1]
    deg = jnp.maximum(jnp.sum(deg_ref[...], axis=0), 1.0)
    agg = aggs / deg[:, None]
    res = jnp.maximum(
        jnp.dot(x_ref[...], ws_ref[...], preferred_element_type=jnp.float32)
        + jnp.dot(agg, wn_ref[...], preferred_element_type=jnp.float32)
        + b_ref[...], 0.0)
    o_ref[...] = res[:N_OUT_K]


def _final(x, agg2, deg32, w_self, w_neigh, b_graph):
    return pl.pallas_call(
        _final_body,
        out_shape=jax.ShapeDtypeStruct((N_OUT_K, GDIM), jnp.float32),
    )(x[:N_PAD_K], agg2, deg32, w_self, w_neigh, b_graph.reshape(1, GDIM))


def kernel(x_customer, x_product, edge_index, lin_c_w, lin_c_b,
           lin_p_w, lin_p_b, w_self, w_neigh, b_graph):
    h_c = _proj(x_customer, lin_c_w, lin_c_b, 1000)
    h_p = _proj(x_product, lin_p_w, lin_p_b, 1000)
    x = jnp.concatenate([h_c, h_p], axis=0)
    agg2, deg32 = _sc_agg(x, edge_index)
    return _final(x, agg2, deg32, w_self, w_neigh, b_graph)


# speedup vs baseline: 10.0249x; 1.0342x over previous
import jax
import jax.numpy as jnp
from jax import lax
from jax.experimental import pallas as pl
from jax.experimental.pallas import tpu as pltpu
from jax.experimental.pallas import tpu_sc as plsc

N_NODES_K = 10000
N_OUT_K = 6000
GDIM = 128
E_K = 320000
NC_K, NS_K = 2, 16
NW_K = NC_K * NS_K
CHUNK_K = 80
EDGES_PER_TILE_K = E_K // NW_K
N_CHUNKS_K = EDGES_PER_TILE_K // CHUNK_K
SP_ROWS_K = 6144
STRIPE_K = SP_ROWS_K // NS_K
TRASH_K = 6016
N_PAD_K = 6016
OUT_STRIPE_K = N_PAD_K // NS_K


_PROJ_BLOCK = 1000


def _proj_body(xc_ref, xp_ref, wc_ref, bc_ref, wp_ref, bp_ref, o_ref):
    i = pl.program_id(0)

    @pl.when(i < 6)
    def _():
        o_ref[...] = jnp.dot(xc_ref[...], wc_ref[...],
                             preferred_element_type=jnp.float32) + bc_ref[...]

    @pl.when(i >= 6)
    def _():
        o_ref[...] = jnp.dot(xp_ref[...], wp_ref[...],
                             preferred_element_type=jnp.float32) + bp_ref[...]


def _proj(x_customer, x_product, lin_c_w, lin_c_b, lin_p_w, lin_p_b):
    full = pl.BlockSpec((GDIM, GDIM), lambda i: (0, 0))
    bias = pl.BlockSpec((1, GDIM), lambda i: (0, 0))
    return pl.pallas_call(
        _proj_body,
        grid=(N_NODES_K // _PROJ_BLOCK,),
        in_specs=[
            pl.BlockSpec((_PROJ_BLOCK, GDIM), lambda i: (jnp.minimum(i, 5), 0)),
            pl.BlockSpec((_PROJ_BLOCK, GDIM), lambda i: (jnp.maximum(i - 6, 0), 0)),
            full, bias, full, bias,
        ],
        out_specs=pl.BlockSpec((_PROJ_BLOCK, GDIM), lambda i: (i, 0)),
        out_shape=jax.ShapeDtypeStruct((N_NODES_K, GDIM), jnp.float32),
    )(x_customer, x_product, lin_c_w, lin_c_b.reshape(1, GDIM),
      lin_p_w, lin_p_b.reshape(1, GDIM))


def _sc_body(x_hbm, src_hbm, dst_hbm, agg_out, deg_out,
             src_b, dst_b, rows0, rows1, deg_v, agg_sh, g0, g1, s0, s1):
    c = lax.axis_index("c")
    s = lax.axis_index("s")
    wid = c * NS_K + s

    pltpu.sync_copy(src_hbm.at[wid], src_b)
    pltpu.sync_copy(dst_hbm.at[wid], dst_b)

    def zero_deg(i, carry):
        deg_v[pl.ds(i * 16, 16)] = jnp.zeros((16,), jnp.float32)
        return carry
    lax.fori_loop(0, SP_ROWS_K // 16, zero_deg, 0)

    def zero_rows(t, carry):
        rows0[t // 8, pl.ds((t % 8) * 16, 16)] = jnp.zeros((16,), jnp.float32)
        return carry
    lax.fori_loop(0, CHUNK_K * 8, zero_rows, 0)
    base_row = s * STRIPE_K
    for j in range(STRIPE_K // CHUNK_K):
        pltpu.sync_copy(rows0, agg_sh.at[pl.ds(base_row + j * CHUNK_K, CHUNK_K)])
    rem = STRIPE_K - (STRIPE_K // CHUNK_K) * CHUNK_K
    pltpu.sync_copy(rows0.at[pl.ds(0, rem)],
                    agg_sh.at[pl.ds(base_row + STRIPE_K - rem, rem)])
    plsc.subcore_barrier()

    ones16 = jnp.ones((16,), jnp.float32)

    def pre_body(t, carry):
        i = t // (CHUNK_K // 16)
        off = (t - i * (CHUNK_K // 16)) * 16
        d16 = dst_b[i, pl.ds(off, 16)]
        d16 = jnp.minimum(d16, TRASH_K)
        dst_b[i, pl.ds(off, 16)] = d16
        plsc.addupdate_scatter(deg_v, [d16], ones16)
        return carry
    lax.fori_loop(0, N_CHUNKS_K * (CHUNK_K // 16), pre_body, 0)

    def chunk_work(i, rows_p, g_p, s_p, rows_o, g_o, s_o):
        pltpu.make_async_copy(x_hbm.at[src_b.at[i]], rows_p, g_p).wait()
        pltpu.async_copy(rows_p, agg_sh.at[dst_b.at[i]], s_p, add=True)

        @pl.when(i >= 1)
        def _():
            pltpu.make_async_copy(rows_o, agg_sh.at[dst_b.at[i]], s_o).wait()

        @pl.when(i + 1 < N_CHUNKS_K)
        def _():
            pltpu.async_copy(x_hbm.at[src_b.at[i + 1]], rows_o, g_o)

    pltpu.async_copy(x_hbm.at[src_b.at[0]], rows0, g0)

    def body(i, carry):
        @pl.when(lax.rem(i, 2) == 0)
        def _():
            chunk_work(i, rows0, g0, s0, rows1, g1, s1)

        @pl.when(lax.rem(i, 2) == 1)
        def _():
            chunk_work(i, rows1, g1, s1, rows0, g0, s0)
        return carry
    lax.fori_loop(0, N_CHUNKS_K, body, 0)
    pltpu.make_async_copy(rows0, agg_sh.at[dst_b.at[0]], s0).wait()
    plsc.subcore_barrier()

    out_base = s * OUT_STRIPE_K
    pltpu.sync_copy(agg_sh.at[pl.ds(out_base, OUT_STRIPE_K)],
                    agg_out.at[c, pl.ds(out_base, OUT_STRIPE_K)])
    pltpu.sync_copy(deg_v.at[pl.ds(0, N_PAD_K)], deg_out.at[wid])


def _sc_agg(x, edge_index):
    mesh = plsc.VectorSubcoreMesh(core_axis_name="c", subcore_axis_name="s")
    idx3 = edge_index.reshape(2, NW_K, N_CHUNKS_K, CHUNK_K)
    return pl.kernel(
        _sc_body,
        out_type=[
            jax.ShapeDtypeStruct((NC_K, N_PAD_K, GDIM), jnp.float32),
            jax.ShapeDtypeStruct((NW_K, N_PAD_K), jnp.float32),
        ],
        mesh=mesh,
        scratch_types=[
            pltpu.VMEM((N_CHUNKS_K, CHUNK_K), jnp.int32),
            pltpu.VMEM((N_CHUNKS_K, CHUNK_K), jnp.int32),
            pltpu.VMEM((CHUNK_K, GDIM), jnp.float32),
            pltpu.VMEM((CHUNK_K, GDIM), jnp.float32),
            pltpu.VMEM((SP_ROWS_K,), jnp.float32),
            pltpu.VMEM_SHARED((SP_ROWS_K, GDIM), jnp.float32),
            pltpu.SemaphoreType.DMA,
            pltpu.SemaphoreType.DMA,
            pltpu.SemaphoreType.DMA,
            pltpu.SemaphoreType.DMA,
        ],
        compiler_params=pltpu.CompilerParams(needs_layout_passes=False),
    )(x, idx3[0], idx3[1])


def _final_body(x_ref, agg_ref, deg_ref, ws_ref, wn_ref, b_ref, o_ref):
    aggs = agg_ref[0] + agg_ref[1]
    deg = jnp.maximum(jnp.sum(deg_ref[...], axis=0), 1.0)
    agg = aggs / deg[:, None]
    res = jnp.maximum(
        jnp.dot(x_ref[...], ws_ref[...], preferred_element_type=jnp.float32)
        + jnp.dot(agg, wn_ref[...], preferred_element_type=jnp.float32)
        + b_ref[...], 0.0)
    o_ref[...] = res[:N_OUT_K]


def _final(x, agg2, deg32, w_self, w_neigh, b_graph):
    whole = lambda shape: pl.BlockSpec(shape, lambda i: tuple(0 for _ in shape))
    return pl.pallas_call(
        _final_body,
        grid=(1,),
        in_specs=[
            pl.BlockSpec((N_PAD_K, GDIM), lambda i: (0, 0)),
            whole((NC_K, N_PAD_K, GDIM)),
            whole((NW_K, N_PAD_K)),
            whole((GDIM, GDIM)),
            whole((GDIM, GDIM)),
            whole((1, GDIM)),
        ],
        out_specs=pl.BlockSpec((N_OUT_K, GDIM), lambda i: (0, 0)),
        out_shape=jax.ShapeDtypeStruct((N_OUT_K, GDIM), jnp.float32),
    )(x, agg2, deg32, w_self, w_neigh, b_graph.reshape(1, GDIM))


def kernel(x_customer, x_product, edge_index, lin_c_w, lin_c_b,
           lin_p_w, lin_p_b, w_self, w_neigh, b_graph):
    x = _proj(x_customer, x_product, lin_c_w, lin_c_b, lin_p_w, lin_p_b)
    agg2, deg32 = _sc_agg(x, edge_index)
    return _final(x, agg2, deg32, w_self, w_neigh, b_graph)
